# Initial kernel scaffold; baseline (speedup 1.0000x reference)
import jax
import jax.numpy as jnp
from jax.experimental import pallas as pl


def kernel(node, X, edge_index, Wi, bi, W_in, b_in, W_h, b_h, W_out, b_out, W_lin, b_lin):
    raise NotImplementedError("write your pallas kernel here")


# speedup vs baseline: 2.6564x; 2.6564x over previous
import functools

import jax
import jax.numpy as jnp
from jax import lax
from jax.experimental import pallas as pl
from jax.experimental.pallas import tpu as pltpu
from jax.experimental.pallas import tpu_sc as plsc

N_NODES = 10000
N_EDGES = 320000
D = 128
NC = 2
NS = 16
RPC = N_NODES // NC
TRASH = RPC
ACC_ROWS = RPC + 8
RPT = 312
ZCH = 104
ZTAIL = ACC_ROWS - NS * RPT
DTAIL = RPC - NS * RPT
EPT = N_EDGES // NS
CH = 80
NCHUNK = EPT // CH
DEG_W = 16


def _make_agg(with_deg: bool):
    mesh = plsc.VectorSubcoreMesh(core_axis_name="c", subcore_axis_name="s")
    out_type = [jax.ShapeDtypeStruct((N_NODES, D), jnp.float32)]
    scratch = [
        pltpu.VMEM((CH,), jnp.int32),
        pltpu.VMEM((CH,), jnp.int32),
        pltpu.VMEM((CH,), jnp.int32),
        pltpu.VMEM((CH, D), jnp.float32),
        pltpu.VMEM((ZCH, D), jnp.float32),
        pltpu.VMEM_SHARED((ACC_ROWS, D), jnp.float32),
        pltpu.SemaphoreType.DMA,
    ]
    if with_deg:
        out_type.append(jax.ShapeDtypeStruct((N_NODES, DEG_W), jnp.float32))
        scratch += [
            pltpu.VMEM((CH, DEG_W), jnp.float32),
            pltpu.VMEM((ZCH, DEG_W), jnp.float32),
            pltpu.VMEM_SHARED((ACC_ROWS, DEG_W), jnp.float32),
        ]

    def body(z_hbm, src_hbm, dst_hbm, *rest):
        if with_deg:
            (out_hbm, deg_hbm, src_v, dst_v, ldst_v, rows_v, zbuf_v, acc_sh,
             sem, ones_v, dbuf_v, deg_sh) = rest
        else:
            (out_hbm, src_v, dst_v, ldst_v, rows_v, zbuf_v, acc_sh,
             sem) = rest
        cid = lax.axis_index("c")
        sid = lax.axis_index("s")

        zeros16 = jnp.zeros((16,), jnp.float32)

        def zrow(r, carry):
            for cc in range(D // 16):
                zbuf_v[r, pl.ds(cc * 16, 16)] = zeros16
            return carry

        lax.fori_loop(0, ZCH, zrow, 0)
        rbase = sid * RPT
        for t in range(RPT // ZCH):
            pltpu.sync_copy(zbuf_v, acc_sh.at[pl.ds(rbase + t * ZCH, ZCH)])

        @pl.when(sid == NS - 1)
        def _():
            pltpu.sync_copy(zbuf_v.at[pl.ds(0, ZTAIL)],
                            acc_sh.at[pl.ds(NS * RPT, ZTAIL)])

        if with_deg:
            ones16 = jnp.ones((16,), jnp.float32)

            def orow(r, carry):
                ones_v[r, pl.ds(0, 16)] = ones16
                return carry

            lax.fori_loop(0, CH, orow, 0)

            def drow(r, carry):
                dbuf_v[r, pl.ds(0, 16)] = zeros16
                return carry

            lax.fori_loop(0, ZCH, drow, 0)
            for t in range(RPT // ZCH):
                pltpu.sync_copy(dbuf_v, deg_sh.at[pl.ds(rbase + t * ZCH, ZCH)])

            @pl.when(sid == NS - 1)
            def _():
                pltpu.sync_copy(dbuf_v.at[pl.ds(0, ZTAIL)],
                                deg_sh.at[pl.ds(NS * RPT, ZTAIL)])

        plsc.subcore_barrier()

        ebase = sid * EPT
        row0 = cid * RPC

        def step(j, carry):
            off = ebase + j * CH
            pltpu.sync_copy(src_hbm.at[pl.ds(off, CH)], src_v)
            pltpu.sync_copy(dst_hbm.at[pl.ds(off, CH)], dst_v)
            gcp = pltpu.async_copy(z_hbm.at[src_v], rows_v, sem)
            for i in range(CH // 16):
                dg = dst_v[pl.ds(i * 16, 16)]
                dl = dg - row0
                inb = (dl >= 0) & (dl < RPC)
                ldst_v[pl.ds(i * 16, 16)] = jnp.where(inb, dl, TRASH)
            gcp.wait()
            pltpu.sync_copy(rows_v, acc_sh.at[ldst_v], add=True)
            if with_deg:
                pltpu.sync_copy(ones_v, deg_sh.at[ldst_v], add=True)
            return carry

        lax.fori_loop(0, NCHUNK, step, 0)

        plsc.subcore_barrier()

        pltpu.sync_copy(acc_sh.at[pl.ds(rbase, RPT)],
                        out_hbm.at[pl.ds(row0 + rbase, RPT)])

        @pl.when(sid == NS - 1)
        def _():
            pltpu.sync_copy(acc_sh.at[pl.ds(NS * RPT, DTAIL)],
                            out_hbm.at[pl.ds(row0 + NS * RPT, DTAIL)])

        if with_deg:
            pltpu.sync_copy(deg_sh.at[pl.ds(rbase, RPT)],
                            deg_hbm.at[pl.ds(row0 + rbase, RPT)])

            @pl.when(sid == NS - 1)
            def _():
                pltpu.sync_copy(deg_sh.at[pl.ds(NS * RPT, DTAIL)],
                                deg_hbm.at[pl.ds(row0 + NS * RPT, DTAIL)])

    return pl.kernel(
        body,
        out_type=tuple(out_type) if with_deg else out_type[0],
        mesh=mesh,
        scratch_types=scratch,
    )


_agg_deg = _make_agg(True)
_agg = _make_agg(False)


_MM_BLK = 1000


def _mm_body(y_ref, w_ref, o_ref):
    o_ref[...] = jnp.dot(y_ref[...], w_ref[...],
                         preferred_element_type=jnp.float32)


def _mm(y, w):
    return pl.pallas_call(
        _mm_body,
        grid=(N_NODES // _MM_BLK,),
        in_specs=[pl.BlockSpec((_MM_BLK, D), lambda i: (i, 0)),
                  pl.BlockSpec((D, D), lambda i: (0, 0))],
        out_specs=pl.BlockSpec((_MM_BLK, D), lambda i: (i, 0)),
        out_shape=jax.ShapeDtypeStruct((N_NODES, D), jnp.float32),
    )(y, w)


def _norm_act(s_ref, deg_ref, b_ref, act):
    s = s_ref[...]
    d = jnp.maximum(deg_ref[:, 0], 1.0)
    y = s / d[:, None] + b_ref[...]
    if act == "elu":
        return jnp.where(y > 0, y, jnp.exp(y) - 1.0)
    return jnp.maximum(y, 0.0)


def _layer_body(act, s_ref, deg_ref, b_ref, w_ref, o_ref):
    y = _norm_act(s_ref, deg_ref, b_ref, act)
    o_ref[...] = jnp.dot(y, w_ref[...], preferred_element_type=jnp.float32)


def _layer(s, deg, b, w, act):
    return pl.pallas_call(
        functools.partial(_layer_body, act),
        grid=(N_NODES // _MM_BLK,),
        in_specs=[pl.BlockSpec((_MM_BLK, D), lambda i: (i, 0)),
                  pl.BlockSpec((_MM_BLK, DEG_W), lambda i: (i, 0)),
                  pl.BlockSpec((1, D), lambda i: (0, 0)),
                  pl.BlockSpec((D, D), lambda i: (0, 0))],
        out_specs=pl.BlockSpec((_MM_BLK, D), lambda i: (i, 0)),
        out_shape=jax.ShapeDtypeStruct((N_NODES, D), jnp.float32),
    )(s, deg, b, w)


def _mean_body(s_ref, deg_ref, b_ref, o_ref):
    i = pl.program_id(0)
    y = _norm_act(s_ref, deg_ref, b_ref, "relu")
    part = jnp.sum(y, axis=0, keepdims=True)

    @pl.when(i == 0)
    def _():
        o_ref[...] = part

    @pl.when(i > 0)
    def _():
        o_ref[...] = o_ref[...] + part


def _mean(s, deg, b):
    return pl.pallas_call(
        _mean_body,
        grid=(N_NODES // _MM_BLK,),
        in_specs=[pl.BlockSpec((_MM_BLK, D), lambda i: (i, 0)),
                  pl.BlockSpec((_MM_BLK, DEG_W), lambda i: (i, 0)),
                  pl.BlockSpec((1, D), lambda i: (0, 0))],
        out_specs=pl.BlockSpec((1, D), lambda i: (0, 0)),
        out_shape=jax.ShapeDtypeStruct((1, D), jnp.float32),
    )(s, deg, b)


def _decode_body(node_ref, wi_ref, bi_ref, ysum_ref, wl_ref, bl_ref, o_ref):
    x = jnp.dot(node_ref[...], wi_ref[...],
                preferred_element_type=jnp.float32) + bi_ref[...]
    w1 = wl_ref[0:D, :]
    w2 = wl_ref[D:2 * D, :]
    ymean = ysum_ref[...] * (1.0 / N_NODES)
    c = jnp.sum(ymean[0, :] * w2[:, 0]) + bl_ref[0, 0]
    o_ref[...] = jnp.dot(x, w1, preferred_element_type=jnp.float32) + c


def _decode(node, wi, bi, ysum, wl, bl):
    batch = node.shape[0]
    return pl.pallas_call(
        _decode_body,
        out_shape=jax.ShapeDtypeStruct((batch, 1), jnp.float32),
    )(node, wi, bi, ysum, wl, bl)


def kernel(node, X, edge_index, Wi, bi, W_in, b_in, W_h, b_h, W_out, b_out,
           W_lin, b_lin):
    src = edge_index[0]
    dst = edge_index[1]
    bi2 = bi.reshape(1, D)
    b_in2 = b_in.reshape(1, D)
    b_h2 = b_h.reshape(1, D)
    b_out2 = b_out.reshape(1, D)
    b_lin2 = b_lin.reshape(1, 1)

    z1 = _mm(X, W_in)
    s1, deg = _agg_deg(z1, src, dst)
    z2 = _layer(s1, deg, b_in2, W_h, "elu")
    s2 = _agg(z2, src, dst)
    z3 = _layer(s2, deg, b_h2, W_out, "relu")
    s3 = _agg(z3, src, dst)
    ysum = _mean(s3, deg, b_out2)
    return _decode(node, Wi, bi2, ysum, W_lin, b_lin2)


# speedup vs baseline: 4.1250x; 1.5529x over previous
import functools

import jax
import jax.numpy as jnp
from jax import lax
from jax.experimental import pallas as pl
from jax.experimental.pallas import tpu as pltpu
from jax.experimental.pallas import tpu_sc as plsc

N_NODES = 10000
N_EDGES = 320000
D = 128
NC = 2
NS = 16
RPC = N_NODES // NC
TRASH = RPC
ACC_ROWS = RPC + 8
RPT = 312
ZCH = 24
ZTAIL = ACC_ROWS - NS * RPT
DTAIL = RPC - NS * RPT
EPT = N_EDGES // NS
CH = 80
NCHUNK = EPT // CH
DEG_W = 16


def _make_agg(with_deg: bool):
    mesh = plsc.VectorSubcoreMesh(core_axis_name="c", subcore_axis_name="s")
    out_type = [jax.ShapeDtypeStruct((N_NODES, D), jnp.float32)]
    scratch = [
        pltpu.VMEM((CH,), jnp.int32),
        pltpu.VMEM((CH,), jnp.int32),
        pltpu.VMEM((CH,), jnp.int32),
        pltpu.VMEM((CH,), jnp.int32),
        pltpu.VMEM((CH,), jnp.int32),
        pltpu.VMEM((CH,), jnp.int32),
        pltpu.VMEM((CH, D), jnp.float32),
        pltpu.VMEM((CH, D), jnp.float32),
        pltpu.VMEM((ZCH, D), jnp.float32),
        pltpu.VMEM_SHARED((ACC_ROWS, D), jnp.float32),
        pltpu.SemaphoreType.DMA,
        pltpu.SemaphoreType.DMA,
    ]
    if with_deg:
        out_type.append(jax.ShapeDtypeStruct((N_NODES, DEG_W), jnp.float32))
        scratch += [
            pltpu.VMEM((CH, DEG_W), jnp.float32),
            pltpu.VMEM((ZCH, DEG_W), jnp.float32),
            pltpu.VMEM_SHARED((ACC_ROWS, DEG_W), jnp.float32),
        ]

    def body(z_hbm, src_hbm, dst_hbm, *rest):
        if with_deg:
            (out_hbm, deg_hbm, src_a, src_b, dst_a, dst_b, ldst_a, ldst_b,
             rows_a, rows_b, zbuf_v, acc_sh, gsem_a, gsem_b,
             ones_v, dbuf_v, deg_sh) = rest
        else:
            (out_hbm, src_a, src_b, dst_a, dst_b, ldst_a, ldst_b,
             rows_a, rows_b, zbuf_v, acc_sh, gsem_a, gsem_b) = rest
        cid = lax.axis_index("c")
        sid = lax.axis_index("s")

        zeros16 = jnp.zeros((16,), jnp.float32)

        def zrow(r, carry):
            for cc in range(D // 16):
                zbuf_v[r, pl.ds(cc * 16, 16)] = zeros16
            return carry

        lax.fori_loop(0, ZCH, zrow, 0)
        rbase = sid * RPT
        for t in range(RPT // ZCH):
            pltpu.sync_copy(zbuf_v, acc_sh.at[pl.ds(rbase + t * ZCH, ZCH)])

        @pl.when(sid == NS - 1)
        def _():
            pltpu.sync_copy(zbuf_v.at[pl.ds(0, ZTAIL)],
                            acc_sh.at[pl.ds(NS * RPT, ZTAIL)])

        if with_deg:
            ones16 = jnp.ones((16,), jnp.float32)

            def orow(r, carry):
                ones_v[r, pl.ds(0, 16)] = ones16
                return carry

            lax.fori_loop(0, CH, orow, 0)

            def drow(r, carry):
                dbuf_v[r, pl.ds(0, 16)] = zeros16
                return carry

            lax.fori_loop(0, ZCH, drow, 0)
            for t in range(RPT // ZCH):
                pltpu.sync_copy(dbuf_v, deg_sh.at[pl.ds(rbase + t * ZCH, ZCH)])

            @pl.when(sid == NS - 1)
            def _():
                pltpu.sync_copy(dbuf_v.at[pl.ds(0, ZTAIL)],
                                deg_sh.at[pl.ds(NS * RPT, ZTAIL)])

        plsc.subcore_barrier()

        ebase = sid * EPT
        row0 = cid * RPC
        srcs = (src_a, src_b)
        dsts = (dst_a, dst_b)
        ldsts = (ldst_a, ldst_b)
        rows = (rows_a, rows_b)
        sems = (gsem_a, gsem_b)

        def load_remap(c, p):
            off = ebase + c * CH
            pltpu.sync_copy(src_hbm.at[pl.ds(off, CH)], srcs[p])
            pltpu.sync_copy(dst_hbm.at[pl.ds(off, CH)], dsts[p])
            for i in range(CH // 16):
                dg = dsts[p][pl.ds(i * 16, 16)]
                dl = dg - row0
                inb = (dl >= 0) & (dl < RPC)
                ldsts[p][pl.ds(i * 16, 16)] = jnp.where(inb, dl, TRASH)

        def gather(p):
            return pltpu.async_copy(z_hbm.at[srcs[p]], rows[p], sems[p])

        def scatter(p):
            pltpu.sync_copy(rows[p], acc_sh.at[ldsts[p]], add=True)
            if with_deg:
                pltpu.sync_copy(ones_v, deg_sh.at[ldsts[p]], add=True)

        W = 5 if with_deg else 10

        def window(w, c2):
            base = w * W
            d = [None] * W
            for p in range(2):
                load_remap(base + p, p)
                d[p] = gather(p)
            for i in range(W):
                d[i].wait()
                scatter(i % 2)
                if i + 2 < W:
                    load_remap(base + i + 2, i % 2)
                    d[i + 2] = gather(i % 2)
            return c2

        lax.fori_loop(0, NCHUNK // W, window, 0)

        plsc.subcore_barrier()

        pltpu.sync_copy(acc_sh.at[pl.ds(rbase, RPT)],
                        out_hbm.at[pl.ds(row0 + rbase, RPT)])

        @pl.when(sid == NS - 1)
        def _():
            pltpu.sync_copy(acc_sh.at[pl.ds(NS * RPT, DTAIL)],
                            out_hbm.at[pl.ds(row0 + NS * RPT, DTAIL)])

        if with_deg:
            pltpu.sync_copy(deg_sh.at[pl.ds(rbase, RPT)],
                            deg_hbm.at[pl.ds(row0 + rbase, RPT)])

            @pl.when(sid == NS - 1)
            def _():
                pltpu.sync_copy(deg_sh.at[pl.ds(NS * RPT, DTAIL)],
                                deg_hbm.at[pl.ds(row0 + NS * RPT, DTAIL)])

    return pl.kernel(
        body,
        out_type=tuple(out_type) if with_deg else out_type[0],
        mesh=mesh,
        scratch_types=scratch,
    )


_agg_deg = _make_agg(True)
_agg = _make_agg(False)


_MM_BLK = 1000


def _mm_body(y_ref, w_ref, o_ref):
    o_ref[...] = jnp.dot(y_ref[...], w_ref[...],
                         preferred_element_type=jnp.float32)


def _mm(y, w):
    return pl.pallas_call(
        _mm_body,
        grid=(N_NODES // _MM_BLK,),
        in_specs=[pl.BlockSpec((_MM_BLK, D), lambda i: (i, 0)),
                  pl.BlockSpec((D, D), lambda i: (0, 0))],
        out_specs=pl.BlockSpec((_MM_BLK, D), lambda i: (i, 0)),
        out_shape=jax.ShapeDtypeStruct((N_NODES, D), jnp.float32),
    )(y, w)


def _norm_act(s_ref, deg_ref, b_ref, act):
    s = s_ref[...]
    d = jnp.maximum(deg_ref[:, 0], 1.0)
    y = s / d[:, None] + b_ref[...]
    if act == "elu":
        return jnp.where(y > 0, y, jnp.exp(y) - 1.0)
    return jnp.maximum(y, 0.0)


def _layer_body(act, s_ref, deg_ref, b_ref, w_ref, o_ref):
    y = _norm_act(s_ref, deg_ref, b_ref, act)
    o_ref[...] = jnp.dot(y, w_ref[...], preferred_element_type=jnp.float32)


def _layer(s, deg, b, w, act):
    return pl.pallas_call(
        functools.partial(_layer_body, act),
        grid=(N_NODES // _MM_BLK,),
        in_specs=[pl.BlockSpec((_MM_BLK, D), lambda i: (i, 0)),
                  pl.BlockSpec((_MM_BLK, DEG_W), lambda i: (i, 0)),
                  pl.BlockSpec((1, D), lambda i: (0, 0)),
                  pl.BlockSpec((D, D), lambda i: (0, 0))],
        out_specs=pl.BlockSpec((_MM_BLK, D), lambda i: (i, 0)),
        out_shape=jax.ShapeDtypeStruct((N_NODES, D), jnp.float32),
    )(s, deg, b, w)


def _mean_body(s_ref, deg_ref, b_ref, o_ref):
    i = pl.program_id(0)
    y = _norm_act(s_ref, deg_ref, b_ref, "relu")
    part = jnp.sum(y, axis=0, keepdims=True)

    @pl.when(i == 0)
    def _():
        o_ref[...] = part

    @pl.when(i > 0)
    def _():
        o_ref[...] = o_ref[...] + part


def _mean(s, deg, b):
    return pl.pallas_call(
        _mean_body,
        grid=(N_NODES // _MM_BLK,),
        in_specs=[pl.BlockSpec((_MM_BLK, D), lambda i: (i, 0)),
                  pl.BlockSpec((_MM_BLK, DEG_W), lambda i: (i, 0)),
                  pl.BlockSpec((1, D), lambda i: (0, 0))],
        out_specs=pl.BlockSpec((1, D), lambda i: (0, 0)),
        out_shape=jax.ShapeDtypeStruct((1, D), jnp.float32),
    )(s, deg, b)


def _decode_body(node_ref, wi_ref, bi_ref, ysum_ref, wl_ref, bl_ref, o_ref):
    x = jnp.dot(node_ref[...], wi_ref[...],
                preferred_element_type=jnp.float32) + bi_ref[...]
    w1 = wl_ref[0:D, :]
    w2 = wl_ref[D:2 * D, :]
    ymean = ysum_ref[...] * (1.0 / N_NODES)
    c = jnp.sum(ymean[0, :] * w2[:, 0]) + bl_ref[0, 0]
    o_ref[...] = jnp.dot(x, w1, preferred_element_type=jnp.float32) + c


def _decode(node, wi, bi, ysum, wl, bl):
    batch = node.shape[0]
    return pl.pallas_call(
        _decode_body,
        out_shape=jax.ShapeDtypeStruct((batch, 1), jnp.float32),
    )(node, wi, bi, ysum, wl, bl)


def kernel(node, X, edge_index, Wi, bi, W_in, b_in, W_h, b_h, W_out, b_out,
           W_lin, b_lin):
    src = edge_index[0]
    dst = edge_index[1]
    bi2 = bi.reshape(1, D)
    b_in2 = b_in.reshape(1, D)
    b_h2 = b_h.reshape(1, D)
    b_out2 = b_out.reshape(1, D)
    b_lin2 = b_lin.reshape(1, 1)

    z1 = _mm(X, W_in)
    s1, deg = _agg_deg(z1, src, dst)
    z2 = _layer(s1, deg, b_in2, W_h, "elu")
    s2 = _agg(z2, src, dst)
    z3 = _layer(s2, deg, b_h2, W_out, "relu")
    s3 = _agg(z3, src, dst)
    ysum = _mean(s3, deg, b_out2)
    return _decode(node, Wi, bi2, ysum, W_lin, b_lin2)


# speedup vs baseline: 5.3604x; 1.2995x over previous
import functools

import jax
import jax.numpy as jnp
from jax import lax
from jax.experimental import pallas as pl
from jax.experimental.pallas import tpu as pltpu
from jax.experimental.pallas import tpu_sc as plsc

N_NODES = 10000
N_EDGES = 320000
D = 128
NC = 2
NS = 16
RPC = N_NODES // NC
TRASH = RPC
ACC_ROWS = RPC + 8
RPT = 312
ZCH = 24
ZTAIL = ACC_ROWS - NS * RPT
DTAIL = RPC - NS * RPT
EPT = N_EDGES // NS
CH = 80
NCHUNK = EPT // CH
DEG_W = 16


def _make_agg(with_deg: bool):
    mesh = plsc.VectorSubcoreMesh(core_axis_name="c", subcore_axis_name="s")
    out_type = [jax.ShapeDtypeStruct((N_NODES, D), jnp.float32)]
    scratch = (
        [pltpu.VMEM((CH,), jnp.int32) for _ in range(4)]
        + [pltpu.VMEM((CH,), jnp.int32) for _ in range(4)]
        + [pltpu.VMEM((CH,), jnp.int32) for _ in range(2)]
        + [pltpu.VMEM((CH, D), jnp.float32) for _ in range(2)]
        + [
            pltpu.VMEM((ZCH, D), jnp.float32),
            pltpu.VMEM_SHARED((ACC_ROWS, D), jnp.float32),
        ]
        + [pltpu.SemaphoreType.DMA for _ in range(4)]
        + [pltpu.SemaphoreType.DMA for _ in range(2)]
    )
    if with_deg:
        out_type.append(jax.ShapeDtypeStruct((N_NODES, DEG_W), jnp.float32))
        scratch += [
            pltpu.VMEM((CH, DEG_W), jnp.float32),
            pltpu.VMEM((ZCH, DEG_W), jnp.float32),
            pltpu.VMEM_SHARED((ACC_ROWS, DEG_W), jnp.float32),
        ]

    def body(z_hbm, src_hbm, dst_hbm, *rest):
        out_hbm = rest[0]
        rest = rest[1:]
        if with_deg:
            deg_hbm = rest[0]
            rest = rest[1:]
        srcs = rest[0:4]
        dsts = rest[4:8]
        ldsts = rest[8:10]
        rows = rest[10:12]
        zbuf_v = rest[12]
        acc_sh = rest[13]
        isems = rest[14:18]
        gsems = rest[18:20]
        if with_deg:
            ones_v, dbuf_v, deg_sh = rest[20:23]
        cid = lax.axis_index("c")
        sid = lax.axis_index("s")

        zeros16 = jnp.zeros((16,), jnp.float32)

        def zrow(r, carry):
            for cc in range(D // 16):
                zbuf_v[r, pl.ds(cc * 16, 16)] = zeros16
            return carry

        lax.fori_loop(0, ZCH, zrow, 0)
        rbase = sid * RPT
        for t in range(RPT // ZCH):
            pltpu.sync_copy(zbuf_v, acc_sh.at[pl.ds(rbase + t * ZCH, ZCH)])

        @pl.when(sid == NS - 1)
        def _():
            pltpu.sync_copy(zbuf_v.at[pl.ds(0, ZTAIL)],
                            acc_sh.at[pl.ds(NS * RPT, ZTAIL)])

        if with_deg:
            ones16 = jnp.ones((16,), jnp.float32)

            def orow(r, carry):
                ones_v[r, pl.ds(0, 16)] = ones16
                return carry

            lax.fori_loop(0, CH, orow, 0)

            def drow(r, carry):
                dbuf_v[r, pl.ds(0, 16)] = zeros16
                return carry

            lax.fori_loop(0, ZCH, drow, 0)
            for t in range(RPT // ZCH):
                pltpu.sync_copy(dbuf_v, deg_sh.at[pl.ds(rbase + t * ZCH, ZCH)])

            @pl.when(sid == NS - 1)
            def _():
                pltpu.sync_copy(dbuf_v.at[pl.ds(0, ZTAIL)],
                                deg_sh.at[pl.ds(NS * RPT, ZTAIL)])

        plsc.subcore_barrier()

        ebase = sid * EPT
        row0 = cid * RPC

        def load_idx(c, q):
            off = ebase + c * CH
            ds = pltpu.async_copy(src_hbm.at[pl.ds(off, CH)], srcs[q],
                                  isems[q])
            dd = pltpu.async_copy(dst_hbm.at[pl.ds(off, CH)], dsts[q],
                                  isems[q])
            return (ds, dd)

        def remap(q, p):
            for i in range(CH // 16):
                dg = dsts[q][pl.ds(i * 16, 16)]
                dl = dg - row0
                inb = (dl >= 0) & (dl < RPC)
                ldsts[p][pl.ds(i * 16, 16)] = jnp.where(inb, dl, TRASH)

        def gather(q, p):
            return pltpu.async_copy(z_hbm.at[srcs[q]], rows[p], gsems[p])

        def scatter(p):
            pltpu.sync_copy(rows[p], acc_sh.at[ldsts[p]], add=True)
            if with_deg:
                pltpu.sync_copy(ones_v, deg_sh.at[ldsts[p]], add=True)

        W = 5 if with_deg else 10

        def window(w, c2):
            base = w * W
            di = [None] * W
            dg = [None] * W
            for q in range(min(4, W)):
                di[q] = load_idx(base + q, q)
            for p in range(2):
                for cp in di[p]:
                    cp.wait()
                remap(p, p)
                dg[p] = gather(p, p)
            for i in range(W):
                dg[i].wait()
                if i + 4 < W:
                    di[i + 4] = load_idx(base + i + 4, i % 4)
                scatter(i % 2)
                if i + 2 < W:
                    for cp in di[i + 2]:
                        cp.wait()
                    remap((i + 2) % 4, i % 2)
                    dg[i + 2] = gather((i + 2) % 4, i % 2)
            return c2

        lax.fori_loop(0, NCHUNK // W, window, 0)

        plsc.subcore_barrier()

        pltpu.sync_copy(acc_sh.at[pl.ds(rbase, RPT)],
                        out_hbm.at[pl.ds(row0 + rbase, RPT)])

        @pl.when(sid == NS - 1)
        def _():
            pltpu.sync_copy(acc_sh.at[pl.ds(NS * RPT, DTAIL)],
                            out_hbm.at[pl.ds(row0 + NS * RPT, DTAIL)])

        if with_deg:
            pltpu.sync_copy(deg_sh.at[pl.ds(rbase, RPT)],
                            deg_hbm.at[pl.ds(row0 + rbase, RPT)])

            @pl.when(sid == NS - 1)
            def _():
                pltpu.sync_copy(deg_sh.at[pl.ds(NS * RPT, DTAIL)],
                                deg_hbm.at[pl.ds(row0 + NS * RPT, DTAIL)])

    return pl.kernel(
        body,
        out_type=tuple(out_type) if with_deg else out_type[0],
        mesh=mesh,
        scratch_types=scratch,
    )


_agg_deg = _make_agg(True)
_agg = _make_agg(False)


_MM_BLK = 1000


def _mm_body(y_ref, w_ref, o_ref):
    o_ref[...] = jnp.dot(y_ref[...], w_ref[...],
                         preferred_element_type=jnp.float32)


def _mm(y, w):
    return pl.pallas_call(
        _mm_body,
        grid=(N_NODES // _MM_BLK,),
        in_specs=[pl.BlockSpec((_MM_BLK, D), lambda i: (i, 0)),
                  pl.BlockSpec((D, D), lambda i: (0, 0))],
        out_specs=pl.BlockSpec((_MM_BLK, D), lambda i: (i, 0)),
        out_shape=jax.ShapeDtypeStruct((N_NODES, D), jnp.float32),
    )(y, w)


def _norm_act(s_ref, deg_ref, b_ref, act):
    s = s_ref[...]
    d = jnp.maximum(deg_ref[:, 0], 1.0)
    y = s / d[:, None] + b_ref[...]
    if act == "elu":
        return jnp.where(y > 0, y, jnp.exp(y) - 1.0)
    return jnp.maximum(y, 0.0)


def _layer_body(act, s_ref, deg_ref, b_ref, w_ref, o_ref):
    y = _norm_act(s_ref, deg_ref, b_ref, act)
    o_ref[...] = jnp.dot(y, w_ref[...], preferred_element_type=jnp.float32)


def _layer(s, deg, b, w, act):
    return pl.pallas_call(
        functools.partial(_layer_body, act),
        grid=(N_NODES // _MM_BLK,),
        in_specs=[pl.BlockSpec((_MM_BLK, D), lambda i: (i, 0)),
                  pl.BlockSpec((_MM_BLK, DEG_W), lambda i: (i, 0)),
                  pl.BlockSpec((1, D), lambda i: (0, 0)),
                  pl.BlockSpec((D, D), lambda i: (0, 0))],
        out_specs=pl.BlockSpec((_MM_BLK, D), lambda i: (i, 0)),
        out_shape=jax.ShapeDtypeStruct((N_NODES, D), jnp.float32),
    )(s, deg, b, w)


def _mean_body(s_ref, deg_ref, b_ref, o_ref):
    i = pl.program_id(0)
    y = _norm_act(s_ref, deg_ref, b_ref, "relu")
    part = jnp.sum(y, axis=0, keepdims=True)

    @pl.when(i == 0)
    def _():
        o_ref[...] = part

    @pl.when(i > 0)
    def _():
        o_ref[...] = o_ref[...] + part


def _mean(s, deg, b):
    return pl.pallas_call(
        _mean_body,
        grid=(N_NODES // _MM_BLK,),
        in_specs=[pl.BlockSpec((_MM_BLK, D), lambda i: (i, 0)),
                  pl.BlockSpec((_MM_BLK, DEG_W), lambda i: (i, 0)),
                  pl.BlockSpec((1, D), lambda i: (0, 0))],
        out_specs=pl.BlockSpec((1, D), lambda i: (0, 0)),
        out_shape=jax.ShapeDtypeStruct((1, D), jnp.float32),
    )(s, deg, b)


def _decode_body(node_ref, wi_ref, bi_ref, ysum_ref, wl_ref, bl_ref, o_ref):
    x = jnp.dot(node_ref[...], wi_ref[...],
                preferred_element_type=jnp.float32) + bi_ref[...]
    w1 = wl_ref[0:D, :]
    w2 = wl_ref[D:2 * D, :]
    ymean = ysum_ref[...] * (1.0 / N_NODES)
    c = jnp.sum(ymean[0, :] * w2[:, 0]) + bl_ref[0, 0]
    o_ref[...] = jnp.dot(x, w1, preferred_element_type=jnp.float32) + c


def _decode(node, wi, bi, ysum, wl, bl):
    batch = node.shape[0]
    return pl.pallas_call(
        _decode_body,
        out_shape=jax.ShapeDtypeStruct((batch, 1), jnp.float32),
    )(node, wi, bi, ysum, wl, bl)


def kernel(node, X, edge_index, Wi, bi, W_in, b_in, W_h, b_h, W_out, b_out,
           W_lin, b_lin):
    src = edge_index[0]
    dst = edge_index[1]
    bi2 = bi.reshape(1, D)
    b_in2 = b_in.reshape(1, D)
    b_h2 = b_h.reshape(1, D)
    b_out2 = b_out.reshape(1, D)
    b_lin2 = b_lin.reshape(1, 1)

    z1 = _mm(X, W_in)
    s1, deg = _agg_deg(z1, src, dst)
    z2 = _layer(s1, deg, b_in2, W_h, "elu")
    s2 = _agg(z2, src, dst)
    z3 = _layer(s2, deg, b_h2, W_out, "relu")
    s3 = _agg(z3, src, dst)
    ysum = _mean(s3, deg, b_out2)
    return _decode(node, Wi, bi2, ysum, W_lin, b_lin2)


# speedup vs baseline: 5.4254x; 1.0121x over previous
import functools

import jax
import jax.numpy as jnp
from jax import lax
from jax.experimental import pallas as pl
from jax.experimental.pallas import tpu as pltpu
from jax.experimental.pallas import tpu_sc as plsc

N_NODES = 10000
N_EDGES = 320000
D = 128
NC = 2
NS = 16
RPC = N_NODES // NC
TRASH = RPC
ACC_ROWS = RPC + 8
RPT = 312
ZCH = 24
ZTAIL = ACC_ROWS - NS * RPT
DTAIL = RPC - NS * RPT
EPT = N_EDGES // NS
CH = 80
NCHUNK = EPT // CH
DEG_W = 16


def _make_agg(with_deg: bool):
    mesh = plsc.VectorSubcoreMesh(core_axis_name="c", subcore_axis_name="s")
    out_type = [jax.ShapeDtypeStruct((N_NODES, D), jnp.float32)]
    scratch = (
        [pltpu.VMEM((CH,), jnp.int32) for _ in range(4)]
        + [pltpu.VMEM((CH,), jnp.int32) for _ in range(4)]
        + [pltpu.VMEM((CH,), jnp.int32) for _ in range(3)]
        + [pltpu.VMEM((CH, D), jnp.float32) for _ in range(3)]
        + [
            pltpu.VMEM((ZCH, D), jnp.float32),
            pltpu.VMEM_SHARED((ACC_ROWS, D), jnp.float32),
        ]
        + [pltpu.SemaphoreType.DMA for _ in range(4)]
        + [pltpu.SemaphoreType.DMA for _ in range(3)]
        + [pltpu.SemaphoreType.DMA for _ in range(3)]
    )
    if with_deg:
        out_type.append(jax.ShapeDtypeStruct((N_NODES, DEG_W), jnp.float32))
        scratch += [
            pltpu.VMEM((CH, DEG_W), jnp.float32),
            pltpu.VMEM((ZCH, DEG_W), jnp.float32),
            pltpu.VMEM_SHARED((ACC_ROWS, DEG_W), jnp.float32),
        ]

    def body(z_hbm, src_hbm, dst_hbm, *rest):
        out_hbm = rest[0]
        rest = rest[1:]
        if with_deg:
            deg_hbm = rest[0]
            rest = rest[1:]
        srcs = rest[0:4]
        dsts = rest[4:8]
        ldsts = rest[8:11]
        rows = rest[11:14]
        zbuf_v = rest[14]
        acc_sh = rest[15]
        isems = rest[16:20]
        gsems = rest[20:23]
        ssems = rest[23:26]
        if with_deg:
            ones_v, dbuf_v, deg_sh = rest[26:29]
        cid = lax.axis_index("c")
        sid = lax.axis_index("s")

        zeros16 = jnp.zeros((16,), jnp.float32)

        def zrow(r, carry):
            for cc in range(D // 16):
                zbuf_v[r, pl.ds(cc * 16, 16)] = zeros16
            return carry

        lax.fori_loop(0, ZCH, zrow, 0)
        rbase = sid * RPT
        for t in range(RPT // ZCH):
            pltpu.sync_copy(zbuf_v, acc_sh.at[pl.ds(rbase + t * ZCH, ZCH)])

        @pl.when(sid == NS - 1)
        def _():
            pltpu.sync_copy(zbuf_v.at[pl.ds(0, ZTAIL)],
                            acc_sh.at[pl.ds(NS * RPT, ZTAIL)])

        if with_deg:
            ones16 = jnp.ones((16,), jnp.float32)

            def orow(r, carry):
                ones_v[r, pl.ds(0, 16)] = ones16
                return carry

            lax.fori_loop(0, CH, orow, 0)

            def drow(r, carry):
                dbuf_v[r, pl.ds(0, 16)] = zeros16
                return carry

            lax.fori_loop(0, ZCH, drow, 0)
            for t in range(RPT // ZCH):
                pltpu.sync_copy(dbuf_v, deg_sh.at[pl.ds(rbase + t * ZCH, ZCH)])

            @pl.when(sid == NS - 1)
            def _():
                pltpu.sync_copy(dbuf_v.at[pl.ds(0, ZTAIL)],
                                deg_sh.at[pl.ds(NS * RPT, ZTAIL)])

        plsc.subcore_barrier()

        ebase = sid * EPT
        row0 = cid * RPC

        def load_idx(c, q):
            off = ebase + c * CH
            ds = pltpu.async_copy(src_hbm.at[pl.ds(off, CH)], srcs[q],
                                  isems[q])
            dd = pltpu.async_copy(dst_hbm.at[pl.ds(off, CH)], dsts[q],
                                  isems[q])
            return (ds, dd)

        def remap(q, p):
            for i in range(CH // 16):
                dg = dsts[q][pl.ds(i * 16, 16)]
                dl = dg - row0
                inb = (dl >= 0) & (dl < RPC)
                ldsts[p][pl.ds(i * 16, 16)] = jnp.where(inb, dl, TRASH)

        def gather(q, p):
            return pltpu.async_copy(z_hbm.at[srcs[q]], rows[p], gsems[p])

        def scatter(p):
            out = [pltpu.async_copy(rows[p], acc_sh.at[ldsts[p]], ssems[p],
                                    add=True)]
            if with_deg:
                out.append(pltpu.async_copy(ones_v, deg_sh.at[ldsts[p]],
                                            ssems[p], add=True))
            return out

        W = 5 if with_deg else 10

        def window(w, c2):
            base = w * W
            di = [None] * W
            dg = [None] * W
            dsc = [None] * W
            for q in range(min(4, W)):
                di[q] = load_idx(base + q, q)
            for p in range(2):
                for cp in di[p]:
                    cp.wait()
                remap(p, p)
                dg[p] = gather(p, p)
            for i in range(W):
                dg[i].wait()
                if i + 4 < W:
                    di[i + 4] = load_idx(base + i + 4, i % 4)
                dsc[i] = scatter(i % 3)
                if i + 2 < W:
                    for cp in di[i + 2]:
                        cp.wait()
                    if i >= 1:
                        for cp in dsc[i - 1]:
                            cp.wait()
                    remap((i + 2) % 4, (i + 2) % 3)
                    dg[i + 2] = gather((i + 2) % 4, (i + 2) % 3)
            for j in range(max(0, W - 3), W):
                for cp in dsc[j]:
                    cp.wait()
            return c2

        lax.fori_loop(0, NCHUNK // W, window, 0)

        plsc.subcore_barrier()

        pltpu.sync_copy(acc_sh.at[pl.ds(rbase, RPT)],
                        out_hbm.at[pl.ds(row0 + rbase, RPT)])

        @pl.when(sid == NS - 1)
        def _():
            pltpu.sync_copy(acc_sh.at[pl.ds(NS * RPT, DTAIL)],
                            out_hbm.at[pl.ds(row0 + NS * RPT, DTAIL)])

        if with_deg:
            pltpu.sync_copy(deg_sh.at[pl.ds(rbase, RPT)],
                            deg_hbm.at[pl.ds(row0 + rbase, RPT)])

            @pl.when(sid == NS - 1)
            def _():
                pltpu.sync_copy(deg_sh.at[pl.ds(NS * RPT, DTAIL)],
                                deg_hbm.at[pl.ds(row0 + NS * RPT, DTAIL)])

    return pl.kernel(
        body,
        out_type=tuple(out_type) if with_deg else out_type[0],
        mesh=mesh,
        scratch_types=scratch,
    )


_agg_deg = _make_agg(True)
_agg = _make_agg(False)


_MM_BLK = 1000


def _mm_body(y_ref, w_ref, o_ref):
    o_ref[...] = jnp.dot(y_ref[...], w_ref[...],
                         preferred_element_type=jnp.float32)


def _mm(y, w):
    return pl.pallas_call(
        _mm_body,
        grid=(N_NODES // _MM_BLK,),
        in_specs=[pl.BlockSpec((_MM_BLK, D), lambda i: (i, 0)),
                  pl.BlockSpec((D, D), lambda i: (0, 0))],
        out_specs=pl.BlockSpec((_MM_BLK, D), lambda i: (i, 0)),
        out_shape=jax.ShapeDtypeStruct((N_NODES, D), jnp.float32),
    )(y, w)


def _norm_act(s_ref, deg_ref, b_ref, act):
    s = s_ref[...]
    d = jnp.maximum(deg_ref[:, 0], 1.0)
    y = s / d[:, None] + b_ref[...]
    if act == "elu":
        return jnp.where(y > 0, y, jnp.exp(y) - 1.0)
    return jnp.maximum(y, 0.0)


def _layer_body(act, s_ref, deg_ref, b_ref, w_ref, o_ref):
    y = _norm_act(s_ref, deg_ref, b_ref, act)
    o_ref[...] = jnp.dot(y, w_ref[...], preferred_element_type=jnp.float32)


def _layer(s, deg, b, w, act):
    return pl.pallas_call(
        functools.partial(_layer_body, act),
        grid=(N_NODES // _MM_BLK,),
        in_specs=[pl.BlockSpec((_MM_BLK, D), lambda i: (i, 0)),
                  pl.BlockSpec((_MM_BLK, DEG_W), lambda i: (i, 0)),
                  pl.BlockSpec((1, D), lambda i: (0, 0)),
                  pl.BlockSpec((D, D), lambda i: (0, 0))],
        out_specs=pl.BlockSpec((_MM_BLK, D), lambda i: (i, 0)),
        out_shape=jax.ShapeDtypeStruct((N_NODES, D), jnp.float32),
    )(s, deg, b, w)


def _mean_body(s_ref, deg_ref, b_ref, o_ref):
    i = pl.program_id(0)
    y = _norm_act(s_ref, deg_ref, b_ref, "relu")
    part = jnp.sum(y, axis=0, keepdims=True)

    @pl.when(i == 0)
    def _():
        o_ref[...] = part

    @pl.when(i > 0)
    def _():
        o_ref[...] = o_ref[...] + part


def _mean(s, deg, b):
    return pl.pallas_call(
        _mean_body,
        grid=(N_NODES // _MM_BLK,),
        in_specs=[pl.BlockSpec((_MM_BLK, D), lambda i: (i, 0)),
                  pl.BlockSpec((_MM_BLK, DEG_W), lambda i: (i, 0)),
                  pl.BlockSpec((1, D), lambda i: (0, 0))],
        out_specs=pl.BlockSpec((1, D), lambda i: (0, 0)),
        out_shape=jax.ShapeDtypeStruct((1, D), jnp.float32),
    )(s, deg, b)


def _decode_body(node_ref, wi_ref, bi_ref, ysum_ref, wl_ref, bl_ref, o_ref):
    x = jnp.dot(node_ref[...], wi_ref[...],
                preferred_element_type=jnp.float32) + bi_ref[...]
    w1 = wl_ref[0:D, :]
    w2 = wl_ref[D:2 * D, :]
    ymean = ysum_ref[...] * (1.0 / N_NODES)
    c = jnp.sum(ymean[0, :] * w2[:, 0]) + bl_ref[0, 0]
    o_ref[...] = jnp.dot(x, w1, preferred_element_type=jnp.float32) + c


def _decode(node, wi, bi, ysum, wl, bl):
    batch = node.shape[0]
    return pl.pallas_call(
        _decode_body,
        out_shape=jax.ShapeDtypeStruct((batch, 1), jnp.float32),
    )(node, wi, bi, ysum, wl, bl)


def kernel(node, X, edge_index, Wi, bi, W_in, b_in, W_h, b_h, W_out, b_out,
           W_lin, b_lin):
    src = edge_index[0]
    dst = edge_index[1]
    bi2 = bi.reshape(1, D)
    b_in2 = b_in.reshape(1, D)
    b_h2 = b_h.reshape(1, D)
    b_out2 = b_out.reshape(1, D)
    b_lin2 = b_lin.reshape(1, 1)

    z1 = _mm(X, W_in)
    s1, deg = _agg_deg(z1, src, dst)
    z2 = _layer(s1, deg, b_in2, W_h, "elu")
    s2 = _agg(z2, src, dst)
    z3 = _layer(s2, deg, b_h2, W_out, "relu")
    s3 = _agg(z3, src, dst)
    ysum = _mean(s3, deg, b_out2)
    return _decode(node, Wi, bi2, ysum, W_lin, b_lin2)
